# baseline (speedup 1.0000x reference)
import functools

import jax
import jax.numpy as jnp
from jax import lax
from jax.experimental import pallas as pl
from jax.experimental.pallas import tpu as pltpu
from jax.experimental.pallas import tpu_sc as plsc

HIDDEN = 64
GAMMA_VAL = 12.0
BATCH_N = 16384
LANES = 16

NUM_CORES = 2
NUM_SUBCORES = 16
NW = NUM_CORES * NUM_SUBCORES
BW = BATCH_N // NW
CHUNK = 128
NCHUNK = BW // CHUNK


def _score_body(hidx, ridx, tidx, ent, rel, out,
                hidx_v, ridx_v, tidx_v, h_v, r_v, t_v, out_v, sem):
    wid = lax.axis_index("s") * NUM_CORES + lax.axis_index("c")
    base = wid * BW

    pltpu.sync_copy(hidx.at[wid], hidx_v)
    pltpu.sync_copy(ridx.at[wid], ridx_v)
    pltpu.sync_copy(tidx.at[wid], tidx_v)

    copies = []
    for c in range(NCHUNK):
        dst = pl.ds(c * CHUNK, CHUNK)
        copies.append(pltpu.async_copy(ent.at[hidx_v.at[c]], h_v.at[dst], sem))
        copies.append(pltpu.async_copy(rel.at[ridx_v.at[c]], r_v.at[dst], sem))
        copies.append(pltpu.async_copy(ent.at[tidx_v.at[c]], t_v.at[dst], sem))
    for cp in copies:
        cp.wait()

    iota = lax.iota(jnp.int32, LANES)

    def body(g, carry):
        row = iota + g * LANES
        acc = jnp.zeros((LANES,), jnp.float32)
        for d in range(HIDDEN):
            col = jnp.full((LANES,), d, jnp.int32)
            h = plsc.load_gather(h_v, [row, col])
            r = plsc.load_gather(r_v, [row, col])
            t = plsc.load_gather(t_v, [row, col])
            acc = acc + jnp.abs(h + r - t)
        out_v[pl.ds(g * LANES, LANES)] = GAMMA_VAL - acc
        return carry

    lax.fori_loop(0, BW // LANES, body, 0)
    pltpu.sync_copy(out_v, out.at[pl.ds(base, BW)])


_sc_call = pl.kernel(
    _score_body,
    out_type=jax.ShapeDtypeStruct((BATCH_N,), jnp.float32),
    mesh=plsc.VectorSubcoreMesh(core_axis_name="c", subcore_axis_name="s"),
    scratch_types=[
        pltpu.VMEM((NCHUNK, CHUNK), jnp.int32),
        pltpu.VMEM((NCHUNK, CHUNK), jnp.int32),
        pltpu.VMEM((NCHUNK, CHUNK), jnp.int32),
        pltpu.VMEM((BW, HIDDEN), jnp.float32),
        pltpu.VMEM((BW, HIDDEN), jnp.float32),
        pltpu.VMEM((BW, HIDDEN), jnp.float32),
        pltpu.VMEM((BW,), jnp.float32),
        pltpu.SemaphoreType.DMA,
    ],
    compiler_params=pltpu.CompilerParams(
        use_tc_tiling_on_sc=False, needs_layout_passes=False
    ),
)


@jax.jit
def kernel(sample, entity_embedding, relation_embedding):
    hidx = sample[:, 0].reshape(NW, NCHUNK, CHUNK)
    ridx = sample[:, 1].reshape(NW, NCHUNK, CHUNK)
    tidx = sample[:, 2].reshape(NW, NCHUNK, CHUNK)
    score = _sc_call(hidx, ridx, tidx, entity_embedding, relation_embedding)
    return score.reshape(BATCH_N, 1)
